# Initial kernel scaffold; baseline (speedup 1.0000x reference)
import jax
import jax.numpy as jnp
from jax.experimental import pallas as pl


def kernel(x, x_3d, pos, edge_index, params):
    raise NotImplementedError("write your pallas kernel here")


# speedup vs baseline: 2.5006x; 2.5006x over previous
import functools

import jax
import jax.numpy as jnp
from jax import lax
from jax.experimental import pallas as pl
from jax.experimental.pallas import tpu as pltpu
from jax.experimental.pallas import tpu_sc as plsc

NG = 50
CUTOFF = 10.0
DIM = 128
ORDER = 5
_DIMS = [DIM // 2 ** i for i in range(ORDER)][::-1]

_NCORES = 2
_NSUB = 16
_LN2 = 0.6931471805599453


def _ssp(v):
    return jnp.maximum(v, 0.0) + jnp.log1p(jnp.exp(-jnp.abs(v))) - _LN2


def _pad32(c):
    return max(32, ((c + 31) // 32) * 32)


def _bs_rows(bn, c):
    return pl.BlockSpec((bn, c), lambda i: (i, 0))


def _bs_full(shape):
    nd = len(shape)
    return pl.BlockSpec(shape, lambda i, _nd=nd: (0,) * _nd)


def _bs_half(bn, c, off):
    return pl.BlockSpec((bn, c), lambda i, _o=off: (i + _o, 0))


def _make_d2(n_nodes, e_pad):
    e_per = e_pad // (_NCORES * _NSUB)
    k2 = 2048
    nblk = e_per // k2
    krow = k2 // 128
    mesh = plsc.VectorSubcoreMesh(core_axis_name="c", subcore_axis_name="s")

    @functools.partial(
        pl.kernel,
        out_type=jax.ShapeDtypeStruct((e_pad // 128, 128), jnp.float32),
        mesh=mesh,
        scratch_types=[
            pltpu.VMEM((n_nodes,), jnp.float32),
            pltpu.VMEM((n_nodes,), jnp.float32),
            pltpu.VMEM((n_nodes,), jnp.float32),
            pltpu.VMEM((krow, 128), jnp.int32),
            pltpu.VMEM((krow, 128), jnp.int32),
            pltpu.VMEM((krow, 128), jnp.float32),
        ],
        compiler_params=pltpu.CompilerParams(
            needs_layout_passes=False, use_tc_tiling_on_sc=False),
    )
    def d2k(px_hbm, py_hbm, pz_hbm, si_hbm, di_hbm, out_hbm,
            px, py, pz, sidx, didx, d2b):
        c = lax.axis_index("c")
        s = lax.axis_index("s")
        wkr = s * _NCORES + c
        pltpu.sync_copy(px_hbm, px)
        pltpu.sync_copy(py_hbm, py)
        pltpu.sync_copy(pz_hbm, pz)
        base_rows0 = wkr * (e_per // 128)

        def blk(b, carry):
            brow = base_rows0 + b * krow
            pltpu.sync_copy(si_hbm.at[pl.ds(brow, krow)], sidx)
            pltpu.sync_copy(di_hbm.at[pl.ds(brow, krow)], didx)

            def grp(j, carry2):
                for t in range(8):
                    sl = pl.ds(t * 16, 16)
                    vs = sidx[j, sl]
                    vd = didx[j, sl]
                    dx = plsc.load_gather(px, [vs]) - plsc.load_gather(px, [vd])
                    dy = plsc.load_gather(py, [vs]) - plsc.load_gather(py, [vd])
                    dz = plsc.load_gather(pz, [vs]) - plsc.load_gather(pz, [vd])
                    d2b[j, sl] = dx * dx + dy * dy + dz * dz
                return carry2

            lax.fori_loop(0, krow, grp, 0)
            pltpu.sync_copy(d2b, out_hbm.at[pl.ds(brow, krow)])
            return carry

        lax.fori_loop(0, nblk, blk, 0)

    return d2k


_GMS_K = {128: 128, 64: 512, 32: 1024, 16: 2048}
_G = 2048


def _make_gms(n_nodes, n_pad, e_pad, hc):
    k = _GMS_K[hc]
    nsb = _G // k
    kb = k // 128
    e_per = e_pad // _NSUB
    nblk = e_per // _G
    g_rows = _G // 128
    rows_per = n_pad // _NSUB
    zr = 32
    nz = rows_per // zr
    mesh = plsc.VectorSubcoreMesh(core_axis_name="c", subcore_axis_name="s")

    @functools.partial(
        pl.kernel,
        out_type=jax.ShapeDtypeStruct((2, n_pad, hc), jnp.float32),
        mesh=mesh,
        scratch_types=[
            pltpu.VMEM((g_rows, 128), jnp.int32),
            pltpu.VMEM((g_rows, 128), jnp.int32),
            pltpu.VMEM((k, hc), jnp.float32),
            pltpu.VMEM((k, hc), jnp.float32),
            pltpu.VMEM((zr, hc), jnp.float32),
            pltpu.VMEM_SHARED((n_pad, hc), jnp.float32),
            pltpu.SemaphoreType.DMA,
        ],
        compiler_params=pltpu.CompilerParams(
            needs_layout_passes=False, use_tc_tiling_on_sc=False),
    )
    def gms(h2_hbm, w2_hbm, si_hbm, di_hbm, out_hbm,
            sidx, didx, rows, wrows, zbuf, agg, sem):
        c = lax.axis_index("c")
        s = lax.axis_index("s")
        coff = c * n_nodes

        def zrow(i, carry):
            for j in range(hc // 16):
                zbuf[i, pl.ds(j * 16, 16)] = jnp.zeros((16,), jnp.float32)
            return carry

        lax.fori_loop(0, zr, zrow, 0)
        r0 = s * rows_per
        for t in range(nz):
            pltpu.sync_copy(zbuf, agg.at[pl.ds(r0 + t * zr, zr)])
        plsc.subcore_barrier()

        base_rows0 = s * (e_per // 128)

        def eblk(b, carry):
            brow = base_rows0 + b * g_rows
            pltpu.sync_copy(si_hbm.at[pl.ds(brow, g_rows)], sidx)
            pltpu.sync_copy(di_hbm.at[pl.ds(brow, g_rows)], didx)

            def shift(j, carry2):
                for t in range(8):
                    sl = pl.ds(t * 16, 16)
                    sidx[j, sl] = sidx[j, sl] + coff
                return carry2

            lax.fori_loop(0, g_rows, shift, 0)
            for sb in range(nsb):
                cps = [
                    pltpu.async_copy(h2_hbm.at[sidx.at[sb * kb + j]],
                                     rows.at[pl.ds(j * 128, 128)], sem)
                    for j in range(kb)
                ]
                pltpu.sync_copy(
                    w2_hbm.at[pl.ds(c * e_pad + s * e_per + b * _G + sb * k,
                                    k)],
                    wrows)
                for cp_ in cps:
                    cp_.wait()

                def mulrow(i, carry2):
                    for j in range(hc // 16):
                        sl = pl.ds(j * 16, 16)
                        rows[i, sl] = rows[i, sl] * wrows[i, sl]
                    return carry2

                lax.fori_loop(0, k, mulrow, 0)
                for j in range(kb):
                    pltpu.sync_copy(rows.at[pl.ds(j * 128, 128)],
                                    agg.at[didx.at[sb * kb + j]], add=True)
            return carry

        lax.fori_loop(0, nblk, eblk, 0)
        plsc.subcore_barrier()
        pltpu.sync_copy(agg.at[pl.ds(r0, rows_per)],
                        out_hbm.at[c, pl.ds(r0, rows_per)])

    return gms


def _filter_w(d2c, wn1, bn1, wn2, bn2, n_edges, hc, be=2048):
    e_pad = d2c.shape[0]
    cp = wn1.shape[1]
    delta = CUTOFF / (NG - 1)
    coeff = -0.5 / delta ** 2

    def body(d2_ref, wn1_ref, bn1_ref, wn2_ref, bn2_ref, out_ref):
        i = pl.program_id(0)
        d = jnp.sqrt(d2_ref[...] + 1e-12)
        offs = delta * lax.broadcasted_iota(jnp.int32, (1, NG), 1).astype(
            jnp.float32)
        diff = d - offs
        rbf = jnp.exp(coeff * (diff * diff))
        u = _ssp(jnp.dot(rbf, wn1_ref[...],
                         preferred_element_type=jnp.float32) + bn1_ref[...])
        w = _ssp(jnp.dot(u, wn2_ref[...],
                         preferred_element_type=jnp.float32) + bn2_ref[...])
        rows = i * be + lax.broadcasted_iota(jnp.int32, (be, 1), 0)
        w = jnp.where(rows < n_edges, w, 0.0)
        out_ref[0] = w[:, :hc]
        out_ref[1] = w[:, hc:]

    return pl.pallas_call(
        body,
        grid=(e_pad // be,),
        in_specs=[
            pl.BlockSpec((be, 1), lambda i: (i, 0)),
            _bs_full((NG, cp)),
            _bs_full((1, cp)),
            _bs_full((cp, cp)),
            _bs_full((1, cp)),
        ],
        out_specs=pl.BlockSpec((2, be, hc), lambda i: (0, i, 0)),
        out_shape=jax.ShapeDtypeStruct((2, e_pad, hc), jnp.float32),
    )(d2c, wn1, bn1, wn2, bn2)


_BN = 2000


def _t0(x3d, w1p, hc):
    n = x3d.shape[0]
    cin = x3d.shape[1]

    def body(x_ref, w_ref, out_ref):
        h = jnp.dot(x_ref[...], w_ref[...], preferred_element_type=jnp.float32)
        out_ref[0] = h[:, :hc]
        out_ref[1] = h[:, hc:]

    return pl.pallas_call(
        body,
        grid=(n // _BN,),
        in_specs=[_bs_rows(_BN, cin), _bs_full(w1p.shape)],
        out_specs=pl.BlockSpec((2, _BN, hc), lambda i: (0, i, 0)),
        out_shape=jax.ShapeDtypeStruct((2, n, hc), jnp.float32),
    )(x3d, w1p)


def _post(agg, hc):
    return [
        pl.BlockSpec((1, _BN, hc), lambda i: (0, i, 0)),
        pl.BlockSpec((1, _BN, hc), lambda i: (1, i, 0)),
    ]


def _t1(agg, w2a, w2b, b2, w1n, hc, hcn, n):

    def body(aa, ab, wa, wb, b2r, w1r, hout, pout):
        o = _ssp(jnp.dot(aa[0], wa[...], preferred_element_type=jnp.float32)
                 + jnp.dot(ab[0], wb[...], preferred_element_type=jnp.float32)
                 + b2r[...])
        pout[...] = o[:, : _DIMS[0]]
        hn = jnp.dot(o[:, _DIMS[0]:], w1r[...],
                     preferred_element_type=jnp.float32)
        hout[0] = hn[:, :hcn]
        hout[1] = hn[:, hcn:]

    return pl.pallas_call(
        body,
        grid=(n // _BN,),
        in_specs=_post(agg, hc) + [
            _bs_full(w2a.shape), _bs_full(w2b.shape), _bs_full(b2.shape),
            _bs_full(w1n.shape),
        ],
        out_specs=[
            pl.BlockSpec((2, _BN, hcn), lambda i: (0, i, 0)),
            _bs_rows(_BN, _DIMS[0]),
        ],
        out_shape=[
            jax.ShapeDtypeStruct((2, n, hcn), jnp.float32),
            jax.ShapeDtypeStruct((n, _DIMS[0]), jnp.float32),
        ],
    )(agg, agg, w2a, w2b, b2, w1n)


def _t2(agg, w2a, w2b, b2, pwa, w1n, hc, hcn, n):
    sd = sum(_DIMS)

    def body(aa, ab, wa, wb, b2r, pr, w1r, hout, dwout):
        o = _ssp(jnp.dot(aa[0], wa[...], preferred_element_type=jnp.float32)
                 + jnp.dot(ab[0], wb[...], preferred_element_type=jnp.float32)
                 + b2r[...])
        dwout[...] = o
        h = pr[...] * o[:, : _DIMS[0]]
        hn = jnp.dot(h, w1r[...], preferred_element_type=jnp.float32)
        hout[0] = hn[:, :hcn]
        hout[1] = hn[:, hcn:]

    return pl.pallas_call(
        body,
        grid=(n // _BN,),
        in_specs=_post(agg, hc) + [
            _bs_full(w2a.shape), _bs_full(w2b.shape), _bs_full(b2.shape),
            _bs_rows(_BN, _DIMS[0]), _bs_full(w1n.shape),
        ],
        out_specs=[
            pl.BlockSpec((2, _BN, hcn), lambda i: (0, i, 0)),
            _bs_rows(_BN, sd),
        ],
        out_shape=[
            jax.ShapeDtypeStruct((2, n, hcn), jnp.float32),
            jax.ShapeDtypeStruct((n, sd), jnp.float32),
        ],
    )(agg, agg, w2a, w2b, b2, pwa, w1n)


def _tmid(agg, w2a, w2b, b2, dw, lo, hi, w1n, hc, hcn, n):
    sd = sum(_DIMS)

    def body(aa, ab, wa, wb, b2r, dwr, w1r, hout):
        o = _ssp(jnp.dot(aa[0], wa[...], preferred_element_type=jnp.float32)
                 + jnp.dot(ab[0], wb[...], preferred_element_type=jnp.float32)
                 + b2r[...])
        h = o * dwr[...][:, lo:hi]
        hn = jnp.dot(h, w1r[...], preferred_element_type=jnp.float32)
        hout[0] = hn[:, :hcn]
        hout[1] = hn[:, hcn:]

    return pl.pallas_call(
        body,
        grid=(n // _BN,),
        in_specs=_post(agg, hc) + [
            _bs_full(w2a.shape), _bs_full(w2b.shape), _bs_full(b2.shape),
            _bs_rows(_BN, sd), _bs_full(w1n.shape),
        ],
        out_specs=pl.BlockSpec((2, _BN, hcn), lambda i: (0, i, 0)),
        out_shape=jax.ShapeDtypeStruct((2, n, hcn), jnp.float32),
    )(agg, agg, w2a, w2b, b2, dw, w1n)


def _t7(agg, w2a, w2b, b2, hc, n):
    cout = w2a.shape[1]

    def body(aa, ab, wa, wb, b2r, out_ref):
        out_ref[...] = _ssp(
            jnp.dot(aa[0], wa[...], preferred_element_type=jnp.float32)
            + jnp.dot(ab[0], wb[...], preferred_element_type=jnp.float32)
            + b2r[...])

    return pl.pallas_call(
        body,
        grid=(n // _BN,),
        in_specs=_post(agg, hc) + [
            _bs_full(w2a.shape), _bs_full(w2b.shape), _bs_full(b2.shape),
        ],
        out_specs=_bs_rows(_BN, cout),
        out_shape=jax.ShapeDtypeStruct((n, cout), jnp.float32),
    )(agg, agg, w2a, w2b, b2)


def _prep_filter(lp, cout):
    cp = _pad32(cout)
    wn1 = jnp.pad(lp["Wn1"], ((0, 0), (0, cp - cout)))
    bn1 = jnp.pad(lp["bn1"], (0, cp - cout)).reshape(1, cp)
    wn2 = jnp.pad(lp["Wn2"], ((0, cp - cout), (0, cp - cout)))
    bn2 = jnp.pad(lp["bn2"], (0, cp - cout)).reshape(1, cp)
    return wn1, bn1, wn2, bn2, cp


def _prep_out(lp, cout):
    cp = _pad32(cout)
    hc = cp // 2
    w2p = jnp.pad(lp["W2"], ((0, cp - cout), (0, 0)))
    return w2p[:hc], w2p[hc:], lp["b2"].reshape(1, cout)


def kernel(x, x_3d, pos, edge_index, params):
    n = x_3d.shape[0]
    e = edge_index.shape[1]
    e_pad = ((e + 32767) // 32768) * 32768

    ei = jnp.pad(edge_index, ((0, 0), (0, e_pad - e)))
    si = ei[0].reshape(e_pad // 128, 128)
    di = ei[1].reshape(e_pad // 128, 128)
    d2 = _make_d2(n, e_pad)(pos[:, 0], pos[:, 1], pos[:, 2], si, di)
    d2c = d2.reshape(e_pad, 1)

    names = ["proj_in", "dwconv", "pw0", "pw1", "pw2", "pw3", "proj_out"]
    couts = [2 * DIM, sum(_DIMS)] + [_DIMS[i + 1] for i in range(ORDER - 1)] + [DIM]
    hcs = [_pad32(c) // 2 for c in couts]
    n_pad = ((n + 2047) // 2048) * 2048
    gms = {}
    for hc in set(hcs):
        gms[hc] = _make_gms(n, n_pad, e_pad, hc)

    def run_edge(layer_idx, h2):
        lp = params[names[layer_idx]]
        hc = hcs[layer_idx]
        wn1, bn1, wn2, bn2, cp = _prep_filter(lp, couts[layer_idx])
        wsp = _filter_w(d2c, wn1, bn1, wn2, bn2, e, hc)
        return gms[hc](h2.reshape(2 * n, hc), wsp.reshape(2 * e_pad, hc),
                       si, di)

    p1 = params["proj_in"]
    w1p = p1["W1"]
    h2 = _t0(x_3d, w1p, hcs[0])
    agg = run_edge(0, h2)

    w2a, w2b, b2 = _prep_out(p1, couts[0])
    pdw = params["dwconv"]
    w1dw = jnp.pad(pdw["W1"], ((0, 0), (0, _pad32(couts[1]) - couts[1])))
    h2, pwa = _t1(agg, w2a, w2b, b2, w1dw, hcs[0], hcs[1], n)
    agg = run_edge(1, h2)

    w2a, w2b, b2 = _prep_out(pdw, couts[1])
    ppw0 = params["pw0"]
    w1n = jnp.pad(ppw0["W1"], ((0, 0), (0, _pad32(couts[2]) - couts[2])))
    h2, dw = _t2(agg, w2a, w2b, b2, pwa, w1n, hcs[1], hcs[2], n)
    agg = run_edge(2, h2)

    bounds = []
    start = 0
    for dcur in _DIMS:
        bounds.append((start, start + dcur))
        start += dcur
    for i in range(3, 6):
        lp_prev = params[names[i - 1]]
        w2a, w2b, b2 = _prep_out(lp_prev, couts[i - 1])
        lp = params[names[i]]
        w1n = jnp.pad(lp["W1"], ((0, 0), (0, _pad32(couts[i]) - couts[i])))
        lo, hi = bounds[i - 2]
        h2 = _tmid(agg, w2a, w2b, b2, dw, lo, hi, w1n, hcs[i - 1], hcs[i], n)
        agg = run_edge(i, h2)

    lp_prev = params["pw3"]
    w2a, w2b, b2 = _prep_out(lp_prev, couts[5])
    lp = params["proj_out"]
    w1n = lp["W1"]
    lo, hi = bounds[4]
    h2 = _tmid(agg, w2a, w2b, b2, dw, lo, hi, w1n, hcs[5], hcs[6], n)
    agg = run_edge(6, h2)

    w2a, w2b, b2 = _prep_out(lp, couts[6])
    return _t7(agg, w2a, w2b, b2, hcs[6], n)
